# Initial kernel scaffold; baseline (speedup 1.0000x reference)
import functools

import jax
import jax.numpy as jnp
from jax import lax
from jax.experimental import pallas as pl
from jax.experimental.pallas import tpu as pltpu
from jax.experimental.pallas import tpu_sc as plsc

SIG = 2048
BATCH = 8192
NC = 2
NS = 16
L = 16
NW = NC * NS
ROWS_PER_W = BATCH // NW
CHUNK = 8
NCHUNK = ROWS_PER_W // CHUNK
NVEC = SIG // L

MAGIC = jnp.float32(12582912.0)

_mesh = plsc.VectorSubcoreMesh(core_axis_name="c", subcore_axis_name="s")


@functools.partial(
    pl.kernel,
    out_type=jax.ShapeDtypeStruct((BATCH, SIG), jnp.float32),
    mesh=_mesh,
    scratch_types=[
        pltpu.VMEM((CHUNK, SIG), jnp.float32),
        pltpu.VMEM((CHUNK, SIG), jnp.float32),
        pltpu.VMEM((ROWS_PER_W, 5), jnp.float32),
    ],
)
def _interp(x_hbm, p_hbm, out_hbm, xbuf, obuf, pbuf):
    wid = lax.axis_index("s") * NC + lax.axis_index("c")
    base = wid * ROWS_PER_W
    pltpu.sync_copy(p_hbm.at[pl.ds(base, ROWS_PER_W)], pbuf)

    iota_i = lax.iota(jnp.int32, L)
    iota_f = iota_i.astype(jnp.float32)
    zeros_i = jnp.zeros((L,), jnp.int32)

    @pl.loop(0, NCHUNK)
    def _chunk(c):
        row0 = base + c * CHUNK
        pltpu.sync_copy(x_hbm.at[pl.ds(row0, CHUNK)], xbuf)
        for r in range(CHUNK):
            prow = c * CHUNK + r
            pr = jnp.broadcast_to(prow, (L,))
            p0 = plsc.load_gather(pbuf, [pr, zeros_i]) * jnp.float32(1e-12)
            p1 = plsc.load_gather(pbuf, [pr, zeros_i + 1]) * jnp.float32(1e-8)
            p2 = plsc.load_gather(pbuf, [pr, zeros_i + 2]) * jnp.float32(1e-4)
            p3 = plsc.load_gather(pbuf, [pr, zeros_i + 3])
            p4 = plsc.load_gather(pbuf, [pr, zeros_i + 4]) * jnp.float32(0.1)
            row_idx = jnp.full((L,), r, jnp.int32)

            @pl.loop(0, NVEC)
            def _vec(j):
                tf = iota_f + (j * L).astype(jnp.float32)
                cv = (((p0 * tf + p1) * tf + p2) * tf + p3) * tf + p4
                ci = (cv + MAGIC) - MAGIC
                k = cv - ci
                posf = jnp.minimum(jnp.maximum(tf - ci, jnp.float32(1.0)),
                                   jnp.float32(2047.0))
                pos = posf.astype(jnp.int32)
                x1 = plsc.load_gather(xbuf, [row_idx, pos])
                x2 = plsc.load_gather(xbuf, [row_idx, pos - 1])
                obuf[r, pl.ds(j * L, L)] = x1 + k * (x2 - x1)

        pltpu.sync_copy(obuf, out_hbm.at[pl.ds(row0, CHUNK)])


def kernel(x, params):
    return _interp(x, params)


# speedup vs baseline: 10.5394x; 10.5394x over previous
import functools

import jax
import jax.numpy as jnp
import numpy as np
from jax import lax
from jax.experimental import pallas as pl
from jax.experimental.pallas import tpu as pltpu
from jax.experimental.pallas import tpu_sc as plsc

SIG = 2048
BATCH = 8192
NC = 2
NS = 16
L = 16
NW = NC * NS
ROWS_PER_W = BATCH // NW
CHUNK = 8
NCHUNK = ROWS_PER_W // CHUNK
NVEC = SIG // L

MAGIC = np.float32(12582912.0)

_mesh = plsc.VectorSubcoreMesh(core_axis_name="c", subcore_axis_name="s")


@functools.partial(
    pl.kernel,
    out_type=jax.ShapeDtypeStruct((BATCH, SIG), jnp.float32),
    mesh=_mesh,
    compiler_params=pltpu.CompilerParams(needs_layout_passes=False),
    scratch_types=[
        pltpu.VMEM((CHUNK, SIG), jnp.float32),
        pltpu.VMEM((CHUNK, SIG), jnp.float32),
        pltpu.VMEM((ROWS_PER_W * 5,), jnp.float32),
    ],
)
def _interp(x_hbm, p_hbm, out_hbm, xbuf, obuf, pbuf):
    wid = lax.axis_index("s") * NC + lax.axis_index("c")
    base = wid * ROWS_PER_W
    pltpu.sync_copy(p_hbm.at[pl.ds(base * 5, ROWS_PER_W * 5)], pbuf)

    iota_i = lax.iota(jnp.int32, L)
    iota_f = iota_i.astype(jnp.float32)
    zeros_i = jnp.zeros((L,), jnp.int32)

    @pl.loop(0, NCHUNK)
    def _chunk(c):
        row0 = base + c * CHUNK
        pltpu.sync_copy(x_hbm.at[pl.ds(row0, CHUNK)], xbuf)
        for r in range(CHUNK):
            prow = c * CHUNK + r
            pr = jnp.broadcast_to(prow * 5, (L,))
            p0 = plsc.load_gather(pbuf, [pr]) / np.float32(1e12)
            p1 = plsc.load_gather(pbuf, [pr + 1]) / np.float32(1e8)
            p2 = plsc.load_gather(pbuf, [pr + 2]) / np.float32(1e4)
            p3 = plsc.load_gather(pbuf, [pr + 3]) / np.float32(1.0)
            p4 = plsc.load_gather(pbuf, [pr + 4]) / np.float32(10.0)
            row_idx = jnp.full((L,), r, jnp.int32)

            @pl.loop(0, NVEC)
            def _vec(j):
                tf = iota_f + (j * L).astype(jnp.float32)
                t2 = tf * tf
                t3 = t2 * tf
                t4 = t2 * t2
                cv = p0 * t4 + p1 * t3 + p2 * t2 + p3 * tf + p4
                ci = (cv + MAGIC) - MAGIC
                k = cv - ci
                posf = jnp.minimum(jnp.maximum(tf - ci, np.float32(1.0)),
                                   np.float32(2047.0))
                pos = posf.astype(jnp.int32)
                x1 = plsc.load_gather(xbuf, [row_idx, pos])
                x2 = plsc.load_gather(xbuf, [row_idx, pos - 1])
                obuf[r, pl.ds(j * L, L)] = x1 + k * (x2 - x1)

        pltpu.sync_copy(obuf, out_hbm.at[pl.ds(row0, CHUNK)])


def kernel(x, params):
    return _interp(x, params.reshape(-1))


# speedup vs baseline: 15.0645x; 1.4293x over previous
import functools

import jax
import jax.numpy as jnp
import numpy as np
from jax import lax
from jax.experimental import pallas as pl
from jax.experimental.pallas import tpu as pltpu
from jax.experimental.pallas import tpu_sc as plsc

SIG = 2048
BATCH = 8192
NC = 2
NS = 16
L = 16
NW = NC * NS
ROWS_PER_W = BATCH // NW
CHUNK = 4
NBUF = 2
NCHUNK = ROWS_PER_W // CHUNK
NVEC = SIG // L

MAGIC = np.float32(12582912.0)

_mesh = plsc.VectorSubcoreMesh(core_axis_name="c", subcore_axis_name="s")


@functools.partial(
    pl.kernel,
    out_type=jax.ShapeDtypeStruct((BATCH, SIG), jnp.float32),
    mesh=_mesh,
    compiler_params=pltpu.CompilerParams(needs_layout_passes=False),
    scratch_types=[
        pltpu.VMEM((NBUF * CHUNK, SIG), jnp.float32),
        pltpu.VMEM((NBUF * CHUNK, SIG), jnp.float32),
        pltpu.VMEM((ROWS_PER_W * 5,), jnp.float32),
        pltpu.SemaphoreType.DMA,
        pltpu.SemaphoreType.DMA,
        pltpu.SemaphoreType.DMA,
        pltpu.SemaphoreType.DMA,
    ],
)
def _interp(x_hbm, p_hbm, out_hbm, xbuf, obuf, pbuf, isem0, isem1, osem0,
            osem1):
    isem = [isem0, isem1]
    osem = [osem0, osem1]
    wid = lax.axis_index("s") * NC + lax.axis_index("c")
    base = wid * ROWS_PER_W
    pltpu.sync_copy(p_hbm.at[pl.ds(base * 5, ROWS_PER_W * 5)], pbuf)

    iota_f = lax.iota(jnp.int32, L).astype(jnp.float32)

    for b in range(NBUF):
        pltpu.async_copy(
            x_hbm.at[pl.ds(base + b * CHUNK, CHUNK)],
            xbuf.at[pl.ds(b * CHUNK, CHUNK)], isem[b])

    @pl.loop(0, NCHUNK, step=NBUF)
    def _outer(g):
        for b in range(NBUF):
            c = g + b
            row0 = base + c * CHUNK
            pltpu.make_async_copy(
                x_hbm.at[pl.ds(row0, CHUNK)],
                xbuf.at[pl.ds(b * CHUNK, CHUNK)], isem[b]).wait()

            @pl.when(c >= NBUF)
            def _():
                pltpu.make_async_copy(
                    obuf.at[pl.ds(b * CHUNK, CHUNK)],
                    out_hbm.at[pl.ds(row0 - NBUF * CHUNK, CHUNK)],
                    osem[b]).wait()

            for r in range(CHUNK):
                pr = jnp.broadcast_to((c * CHUNK + r) * 5, (L,))
                p0 = plsc.load_gather(pbuf, [pr])
                p1 = plsc.load_gather(pbuf, [pr + 1])
                p2 = plsc.load_gather(pbuf, [pr + 2])
                p3 = plsc.load_gather(pbuf, [pr + 3])
                p4 = plsc.load_gather(pbuf, [pr + 4])
                row_idx = jnp.full((L,), b * CHUNK + r, jnp.int32)

                @plsc.parallel_loop(0, NVEC, unroll=4)
                def _vec(j):
                    tf = iota_f + (j * L).astype(jnp.float32)
                    t2 = tf * tf
                    t3 = t2 * tf
                    t4 = t2 * t2
                    cv = p0 * t4 + p1 * t3 + p2 * t2 + p3 * tf + p4
                    ci = (cv + MAGIC) - MAGIC
                    k = cv - ci
                    posf = jnp.minimum(
                        jnp.maximum(tf - ci, np.float32(1.0)),
                        np.float32(2047.0))
                    pos = posf.astype(jnp.int32)
                    x1 = plsc.load_gather(xbuf, [row_idx, pos])
                    x2 = plsc.load_gather(xbuf, [row_idx, pos - 1])
                    obuf[b * CHUNK + r, pl.ds(j * L, L)] = x1 + k * (x2 - x1)

            pltpu.async_copy(
                obuf.at[pl.ds(b * CHUNK, CHUNK)],
                out_hbm.at[pl.ds(row0, CHUNK)], osem[b])

            @pl.when(c + NBUF < NCHUNK)
            def _():
                pltpu.async_copy(
                    x_hbm.at[pl.ds(row0 + NBUF * CHUNK, CHUNK)],
                    xbuf.at[pl.ds(b * CHUNK, CHUNK)], isem[b])

    for b in range(NBUF):
        pltpu.make_async_copy(
            obuf.at[pl.ds(b * CHUNK, CHUNK)],
            out_hbm.at[pl.ds(base + b * CHUNK, CHUNK)], osem[b]).wait()


def kernel(x, params):
    scaler = jnp.array([1.0e12, 1.0e8, 1.0e4, 1.0, 10.0], dtype=jnp.float32)
    return _interp(x, (params / scaler).reshape(-1))


# speedup vs baseline: 16.6295x; 1.1039x over previous
import functools

import jax
import jax.numpy as jnp
import numpy as np
from jax import lax
from jax.experimental import pallas as pl
from jax.experimental.pallas import tpu as pltpu
from jax.experimental.pallas import tpu_sc as plsc

SIG = 2048
BATCH = 8192
NC = 2
NS = 16
L = 16
NW = NC * NS
ROWS_PER_W = BATCH // NW
CHUNK = 4
NBUF = 2
NCHUNK = ROWS_PER_W // CHUNK
NVEC = SIG // L

MAGIC = np.float32(12582912.0)

_mesh = plsc.VectorSubcoreMesh(core_axis_name="c", subcore_axis_name="s")


@functools.partial(
    pl.kernel,
    out_type=jax.ShapeDtypeStruct((BATCH, SIG), jnp.float32),
    mesh=_mesh,
    compiler_params=pltpu.CompilerParams(needs_layout_passes=False),
    scratch_types=[
        pltpu.VMEM((NBUF * CHUNK, SIG), jnp.float32),
        pltpu.VMEM((NBUF * CHUNK, SIG), jnp.float32),
        pltpu.VMEM((ROWS_PER_W * 5,), jnp.float32),
        pltpu.VMEM((SIG,), jnp.float32),
        pltpu.VMEM((SIG,), jnp.float32),
        pltpu.VMEM((SIG,), jnp.float32),
        pltpu.VMEM((SIG,), jnp.float32),
        pltpu.SemaphoreType.DMA,
        pltpu.SemaphoreType.DMA,
        pltpu.SemaphoreType.DMA,
        pltpu.SemaphoreType.DMA,
    ],
)
def _interp(x_hbm, p_hbm, out_hbm, xbuf, obuf, pbuf, tb1, tb2, tb3, tb4,
            isem0, isem1, osem0, osem1):
    isem = [isem0, isem1]
    osem = [osem0, osem1]
    wid = lax.axis_index("s") * NC + lax.axis_index("c")
    base = wid * ROWS_PER_W
    pltpu.sync_copy(p_hbm.at[pl.ds(base * 5, ROWS_PER_W * 5)], pbuf)

    iota_f = lax.iota(jnp.int32, L).astype(jnp.float32)

    @pl.loop(0, NVEC)
    def _tab(j):
        tf = iota_f + (j * L).astype(jnp.float32)
        t2 = tf * tf
        tb1[pl.ds(j * L, L)] = tf
        tb2[pl.ds(j * L, L)] = t2
        tb3[pl.ds(j * L, L)] = t2 * tf
        tb4[pl.ds(j * L, L)] = t2 * t2

    for b in range(NBUF):
        pltpu.async_copy(
            x_hbm.at[pl.ds(base + b * CHUNK, CHUNK)],
            xbuf.at[pl.ds(b * CHUNK, CHUNK)], isem[b])

    @pl.loop(0, NCHUNK, step=NBUF)
    def _outer(g):
        for b in range(NBUF):
            c = g + b
            row0 = base + c * CHUNK
            pltpu.make_async_copy(
                x_hbm.at[pl.ds(row0, CHUNK)],
                xbuf.at[pl.ds(b * CHUNK, CHUNK)], isem[b]).wait()

            @pl.when(c >= NBUF)
            def _():
                pltpu.make_async_copy(
                    obuf.at[pl.ds(b * CHUNK, CHUNK)],
                    out_hbm.at[pl.ds(row0 - NBUF * CHUNK, CHUNK)],
                    osem[b]).wait()

            for r in range(CHUNK):
                pr = jnp.broadcast_to((c * CHUNK + r) * 5, (L,))
                p0 = plsc.load_gather(pbuf, [pr])
                p1 = plsc.load_gather(pbuf, [pr + 1])
                p2 = plsc.load_gather(pbuf, [pr + 2])
                p3 = plsc.load_gather(pbuf, [pr + 3])
                p4 = plsc.load_gather(pbuf, [pr + 4])
                row_idx = jnp.full((L,), b * CHUNK + r, jnp.int32)

                @plsc.parallel_loop(0, NVEC, unroll=4)
                def _vec(j):
                    sl = pl.ds(j * L, L)
                    tf = tb1[sl]
                    t2 = tb2[sl]
                    t3 = tb3[sl]
                    t4 = tb4[sl]
                    cv = p0 * t4 + p1 * t3 + p2 * t2 + p3 * tf + p4
                    ci = (cv + MAGIC) - MAGIC
                    k = cv - ci
                    posf = jnp.minimum(
                        jnp.maximum(tf - ci, np.float32(1.0)),
                        np.float32(2047.0))
                    pos = posf.astype(jnp.int32)
                    x1 = plsc.load_gather(xbuf, [row_idx, pos])
                    x2 = plsc.load_gather(xbuf, [row_idx, pos - 1])
                    obuf[b * CHUNK + r, pl.ds(j * L, L)] = x1 + k * (x2 - x1)

            pltpu.async_copy(
                obuf.at[pl.ds(b * CHUNK, CHUNK)],
                out_hbm.at[pl.ds(row0, CHUNK)], osem[b])

            @pl.when(c + NBUF < NCHUNK)
            def _():
                pltpu.async_copy(
                    x_hbm.at[pl.ds(row0 + NBUF * CHUNK, CHUNK)],
                    xbuf.at[pl.ds(b * CHUNK, CHUNK)], isem[b])

    for b in range(NBUF):
        pltpu.make_async_copy(
            obuf.at[pl.ds(b * CHUNK, CHUNK)],
            out_hbm.at[pl.ds(base + b * CHUNK, CHUNK)], osem[b]).wait()


def kernel(x, params):
    scaler = jnp.array([1.0e12, 1.0e8, 1.0e4, 1.0, 10.0], dtype=jnp.float32)
    return _interp(x, (params / scaler).reshape(-1))


# speedup vs baseline: 18.2690x; 1.0986x over previous
import functools

import jax
import jax.numpy as jnp
import numpy as np
from jax import lax
from jax.experimental import pallas as pl
from jax.experimental.pallas import tpu as pltpu
from jax.experimental.pallas import tpu_sc as plsc

SIG = 2048
BATCH = 8192
NC = 2
NS = 16
L = 16
NW = NC * NS
ROWS_PER_W = BATCH // NW
CHUNK = 4
NBUF = 2
NCHUNK = ROWS_PER_W // CHUNK
NVEC = SIG // L

MAGIC = np.float32(12582912.0)

_mesh = plsc.VectorSubcoreMesh(core_axis_name="c", subcore_axis_name="s")


@functools.partial(
    pl.kernel,
    out_type=jax.ShapeDtypeStruct((BATCH, SIG), jnp.float32),
    mesh=_mesh,
    compiler_params=pltpu.CompilerParams(needs_layout_passes=False),
    scratch_types=[
        pltpu.VMEM((NBUF * CHUNK, SIG), jnp.float32),
        pltpu.VMEM((NBUF * CHUNK, SIG), jnp.float32),
        pltpu.VMEM((ROWS_PER_W * 5,), jnp.float32),
        pltpu.VMEM((SIG,), jnp.float32),
        pltpu.VMEM((SIG,), jnp.float32),
        pltpu.VMEM((SIG,), jnp.float32),
        pltpu.VMEM((SIG,), jnp.float32),
        pltpu.VMEM((SIG,), jnp.int32),
        pltpu.SemaphoreType.DMA,
        pltpu.SemaphoreType.DMA,
        pltpu.SemaphoreType.DMA,
        pltpu.SemaphoreType.DMA,
    ],
)
def _interp(x_hbm, p_hbm, out_hbm, xbuf, obuf, pbuf, tb1, tb2, tb3, tb4,
            toff, isem0, isem1, osem0, osem1):
    isem = [isem0, isem1]
    osem = [osem0, osem1]
    wid = lax.axis_index("s") * NC + lax.axis_index("c")
    base = wid * ROWS_PER_W
    pltpu.sync_copy(p_hbm.at[pl.ds(base * 5, ROWS_PER_W * 5)], pbuf)

    iota_f = lax.iota(jnp.int32, L).astype(jnp.float32)

    iota_i = lax.iota(jnp.int32, L)

    @pl.loop(0, NVEC)
    def _tab(j):
        tf = iota_f + (j * L).astype(jnp.float32)
        t2 = tf * tf
        tb1[pl.ds(j * L, L)] = tf
        tb2[pl.ds(j * L, L)] = t2
        tb3[pl.ds(j * L, L)] = t2 * tf
        tb4[pl.ds(j * L, L)] = t2 * t2
        toff[pl.ds(j * L, L)] = iota_i + (j * L + 0x400000)

    for b in range(NBUF):
        pltpu.async_copy(
            x_hbm.at[pl.ds(base + b * CHUNK, CHUNK)],
            xbuf.at[pl.ds(b * CHUNK, CHUNK)], isem[b])

    @pl.loop(0, NCHUNK, step=NBUF)
    def _outer(g):
        for b in range(NBUF):
            c = g + b
            row0 = base + c * CHUNK
            pltpu.make_async_copy(
                x_hbm.at[pl.ds(row0, CHUNK)],
                xbuf.at[pl.ds(b * CHUNK, CHUNK)], isem[b]).wait()

            @pl.when(c >= NBUF)
            def _():
                pltpu.make_async_copy(
                    obuf.at[pl.ds(b * CHUNK, CHUNK)],
                    out_hbm.at[pl.ds(row0 - NBUF * CHUNK, CHUNK)],
                    osem[b]).wait()

            for rp in range(CHUNK // 2):
                rows = (rp * 2, rp * 2 + 1)
                coeffs = []
                for r in rows:
                    pr = jnp.broadcast_to((c * CHUNK + r) * 5, (L,))
                    coeffs.append(tuple(
                        plsc.load_gather(pbuf, [pr + i]) for i in range(5)))

                @plsc.parallel_loop(0, NVEC, unroll=2)
                def _vec(j):
                    sl = pl.ds(j * L, L)
                    tf = tb1[sl]
                    t2 = tb2[sl]
                    t3 = tb3[sl]
                    t4 = tb4[sl]
                    to = toff[sl]
                    for r, (p0, p1, p2, p3, p4) in zip(rows, coeffs):
                        row_idx = jnp.full((L,), b * CHUNK + r, jnp.int32)
                        cv = p0 * t4 + p1 * t3 + p2 * t2 + p3 * tf + p4
                        cvp = cv + MAGIC
                        ci = cvp - MAGIC
                        k = cv - ci
                        bits = lax.bitcast_convert_type(cvp, jnp.int32)
                        pos = jnp.minimum(
                            jnp.maximum(to - (bits & 0x7FFFFF), 1), 2047)
                        x1 = plsc.load_gather(xbuf, [row_idx, pos])
                        x2 = plsc.load_gather(xbuf, [row_idx, pos - 1])
                        obuf[b * CHUNK + r, sl] = x1 + k * (x2 - x1)

            pltpu.async_copy(
                obuf.at[pl.ds(b * CHUNK, CHUNK)],
                out_hbm.at[pl.ds(row0, CHUNK)], osem[b])

            @pl.when(c + NBUF < NCHUNK)
            def _():
                pltpu.async_copy(
                    x_hbm.at[pl.ds(row0 + NBUF * CHUNK, CHUNK)],
                    xbuf.at[pl.ds(b * CHUNK, CHUNK)], isem[b])

    for b in range(NBUF):
        pltpu.make_async_copy(
            obuf.at[pl.ds(b * CHUNK, CHUNK)],
            out_hbm.at[pl.ds(base + b * CHUNK, CHUNK)], osem[b]).wait()


def kernel(x, params):
    scaler = jnp.array([1.0e12, 1.0e8, 1.0e4, 1.0, 10.0], dtype=jnp.float32)
    return _interp(x, (params / scaler).reshape(-1))


# speedup vs baseline: 18.2815x; 1.0007x over previous
import functools

import jax
import jax.numpy as jnp
import numpy as np
from jax import lax
from jax.experimental import pallas as pl
from jax.experimental.pallas import tpu as pltpu
from jax.experimental.pallas import tpu_sc as plsc

SIG = 2048
BATCH = 8192
NC = 2
NS = 16
L = 16
NW = NC * NS
ROWS_PER_W = BATCH // NW
CHUNK = 4
NBUF = 2
NCHUNK = ROWS_PER_W // CHUNK
NVEC = SIG // L

MAGIC = np.float32(12582912.0)

_mesh = plsc.VectorSubcoreMesh(core_axis_name="c", subcore_axis_name="s")


@functools.partial(
    pl.kernel,
    out_type=jax.ShapeDtypeStruct((BATCH, SIG), jnp.float32),
    mesh=_mesh,
    compiler_params=pltpu.CompilerParams(needs_layout_passes=False),
    scratch_types=[
        pltpu.VMEM((NBUF * CHUNK, SIG), jnp.float32),
        pltpu.VMEM((NBUF * CHUNK, SIG), jnp.float32),
        pltpu.VMEM((ROWS_PER_W * 5,), jnp.float32),
        pltpu.VMEM((SIG,), jnp.float32),
        pltpu.VMEM((SIG,), jnp.float32),
        pltpu.VMEM((SIG,), jnp.float32),
        pltpu.VMEM((SIG,), jnp.float32),
        pltpu.VMEM((SIG,), jnp.int32),
        pltpu.SemaphoreType.DMA,
        pltpu.SemaphoreType.DMA,
        pltpu.SemaphoreType.DMA,
        pltpu.SemaphoreType.DMA,
    ],
)
def _interp(x_hbm, p_hbm, out_hbm, xbuf, obuf, pbuf, tb1, tb2, tb3, tb4,
            toff, isem0, isem1, osem0, osem1):
    isem = [isem0, isem1]
    osem = [osem0, osem1]
    wid = lax.axis_index("s") * NC + lax.axis_index("c")
    base = wid * ROWS_PER_W
    pltpu.sync_copy(p_hbm.at[pl.ds(base * 5, ROWS_PER_W * 5)], pbuf)

    iota_f = lax.iota(jnp.int32, L).astype(jnp.float32)

    iota_i = lax.iota(jnp.int32, L)

    @pl.loop(0, NVEC)
    def _tab(j):
        tf = iota_f + (j * L).astype(jnp.float32)
        t2 = tf * tf
        tb1[pl.ds(j * L, L)] = tf
        tb2[pl.ds(j * L, L)] = t2
        tb3[pl.ds(j * L, L)] = t2 * tf
        tb4[pl.ds(j * L, L)] = t2 * t2
        toff[pl.ds(j * L, L)] = iota_i + (j * L + 0x400000)

    for b in range(NBUF):
        pltpu.async_copy(
            x_hbm.at[pl.ds(base + b * CHUNK, CHUNK)],
            xbuf.at[pl.ds(b * CHUNK, CHUNK)], isem[b])

    @pl.loop(0, NCHUNK, step=NBUF)
    def _outer(g):
        for b in range(NBUF):
            c = g + b
            row0 = base + c * CHUNK
            pltpu.make_async_copy(
                x_hbm.at[pl.ds(row0, CHUNK)],
                xbuf.at[pl.ds(b * CHUNK, CHUNK)], isem[b]).wait()

            @pl.when(c >= NBUF)
            def _():
                pltpu.make_async_copy(
                    obuf.at[pl.ds(b * CHUNK, CHUNK)],
                    out_hbm.at[pl.ds(row0 - NBUF * CHUNK, CHUNK)],
                    osem[b]).wait()

            if True:
                rows = tuple(range(CHUNK))
                coeffs = []
                for r in rows:
                    pr = jnp.broadcast_to((c * CHUNK + r) * 5, (L,))
                    coeffs.append(tuple(
                        plsc.load_gather(pbuf, [pr + i]) for i in range(5)))

                @plsc.parallel_loop(0, NVEC, unroll=1)
                def _vec(j):
                    sl = pl.ds(j * L, L)
                    tf = tb1[sl]
                    t2 = tb2[sl]
                    t3 = tb3[sl]
                    t4 = tb4[sl]
                    to = toff[sl]
                    for r, (p0, p1, p2, p3, p4) in zip(rows, coeffs):
                        row_idx = jnp.full((L,), b * CHUNK + r, jnp.int32)
                        cv = p0 * t4 + p1 * t3 + p2 * t2 + p3 * tf + p4
                        cvp = cv + MAGIC
                        ci = cvp - MAGIC
                        k = cv - ci
                        bits = lax.bitcast_convert_type(cvp, jnp.int32)
                        pos = jnp.minimum(
                            jnp.maximum(to - (bits & 0x7FFFFF), 1), 2047)
                        x1 = plsc.load_gather(xbuf, [row_idx, pos])
                        x2 = plsc.load_gather(xbuf, [row_idx, pos - 1])
                        obuf[b * CHUNK + r, sl] = x1 + k * (x2 - x1)

            pltpu.async_copy(
                obuf.at[pl.ds(b * CHUNK, CHUNK)],
                out_hbm.at[pl.ds(row0, CHUNK)], osem[b])

            @pl.when(c + NBUF < NCHUNK)
            def _():
                pltpu.async_copy(
                    x_hbm.at[pl.ds(row0 + NBUF * CHUNK, CHUNK)],
                    xbuf.at[pl.ds(b * CHUNK, CHUNK)], isem[b])

    for b in range(NBUF):
        pltpu.make_async_copy(
            obuf.at[pl.ds(b * CHUNK, CHUNK)],
            out_hbm.at[pl.ds(base + b * CHUNK, CHUNK)], osem[b]).wait()


def kernel(x, params):
    scaler = jnp.array([1.0e12, 1.0e8, 1.0e4, 1.0, 10.0], dtype=jnp.float32)
    return _interp(x, (params / scaler).reshape(-1))


# speedup vs baseline: 18.2929x; 1.0006x over previous
import functools

import jax
import jax.numpy as jnp
import numpy as np
from jax import lax
from jax.experimental import pallas as pl
from jax.experimental.pallas import tpu as pltpu
from jax.experimental.pallas import tpu_sc as plsc

SIG = 2048
BATCH = 8192
NC = 2
NS = 16
L = 16
NW = NC * NS
ROWS_PER_W = BATCH // NW
CHUNK = 4
NBUF = 2
NCHUNK = ROWS_PER_W // CHUNK
NVEC = SIG // L

MAGIC = np.float32(12582912.0)

_mesh = plsc.VectorSubcoreMesh(core_axis_name="c", subcore_axis_name="s")


@functools.partial(
    pl.kernel,
    out_type=jax.ShapeDtypeStruct((BATCH, SIG), jnp.float32),
    mesh=_mesh,
    compiler_params=pltpu.CompilerParams(needs_layout_passes=False),
    scratch_types=[
        pltpu.VMEM((NBUF * CHUNK, SIG), jnp.float32),
        pltpu.VMEM((NBUF * CHUNK, SIG), jnp.float32),
        pltpu.VMEM((ROWS_PER_W * 5,), jnp.float32),
        pltpu.VMEM((SIG,), jnp.float32),
        pltpu.VMEM((SIG,), jnp.float32),
        pltpu.VMEM((SIG,), jnp.float32),
        pltpu.VMEM((SIG,), jnp.float32),
        pltpu.VMEM((SIG,), jnp.int32),
        pltpu.SemaphoreType.DMA,
        pltpu.SemaphoreType.DMA,
        pltpu.SemaphoreType.DMA,
        pltpu.SemaphoreType.DMA,
    ],
)
def _interp(x_hbm, p_hbm, out_hbm, xbuf, obuf, pbuf, tb1, tb2, tb3, tb4,
            toff, isem0, isem1, osem0, osem1):
    isem = [isem0, isem1]
    osem = [osem0, osem1]
    wid = lax.axis_index("s") * NC + lax.axis_index("c")
    base = wid * ROWS_PER_W
    pltpu.sync_copy(p_hbm.at[pl.ds(base * 5, ROWS_PER_W * 5)], pbuf)

    iota_f = lax.iota(jnp.int32, L).astype(jnp.float32)

    iota_i = lax.iota(jnp.int32, L)

    @pl.loop(0, NVEC)
    def _tab(j):
        tf = iota_f + (j * L).astype(jnp.float32)
        t2 = tf * tf
        tb1[pl.ds(j * L, L)] = tf
        tb2[pl.ds(j * L, L)] = t2
        tb3[pl.ds(j * L, L)] = t2 * tf
        tb4[pl.ds(j * L, L)] = t2 * t2
        toff[pl.ds(j * L, L)] = iota_i + (j * L + 0x400000)

    for b in range(NBUF):
        pltpu.async_copy(
            x_hbm.at[pl.ds(base + b * CHUNK, CHUNK)],
            xbuf.at[pl.ds(b * CHUNK, CHUNK)], isem[b])

    @pl.loop(0, NCHUNK, step=NBUF)
    def _outer(g):
        for b in range(NBUF):
            c = g + b
            row0 = base + c * CHUNK
            pltpu.make_async_copy(
                x_hbm.at[pl.ds(row0, CHUNK)],
                xbuf.at[pl.ds(b * CHUNK, CHUNK)], isem[b]).wait()

            @pl.when(c >= NBUF)
            def _():
                pltpu.make_async_copy(
                    obuf.at[pl.ds(b * CHUNK, CHUNK)],
                    out_hbm.at[pl.ds(row0 - NBUF * CHUNK, CHUNK)],
                    osem[b]).wait()

            coeffs = []
            for r in range(CHUNK):
                pr = jnp.broadcast_to((c * CHUNK + r) * 5, (L,))
                coeffs.append(tuple(
                    plsc.load_gather(pbuf, [pr + i]) for i in range(5)))

            @plsc.parallel_loop(0, NVEC, unroll=1)
            def _vec(j):
                sl = pl.ds(j * L, L)
                tf = tb1[sl]
                t2 = tb2[sl]
                t3 = tb3[sl]
                t4 = tb4[sl]
                to = toff[sl]
                for r, (p0, p1, p2, p3, p4) in enumerate(coeffs):
                    row_idx = jnp.full((L,), b * CHUNK + r, jnp.int32)
                    cv = p0 * t4 + p1 * t3 + p2 * t2 + p3 * tf + p4
                    cvp = cv + MAGIC
                    ci = cvp - MAGIC
                    k = cv - ci
                    bits = lax.bitcast_convert_type(cvp, jnp.int32)
                    pos = jnp.minimum(
                        jnp.maximum(to - (bits & 0x7FFFFF), 1), 2047)
                    x1 = plsc.load_gather(xbuf, [row_idx, pos])
                    x2 = plsc.load_gather(xbuf, [row_idx, pos - 1])
                    obuf[b * CHUNK + r, sl] = x1 + k * (x2 - x1)

            pltpu.async_copy(
                obuf.at[pl.ds(b * CHUNK, CHUNK)],
                out_hbm.at[pl.ds(row0, CHUNK)], osem[b])

            @pl.when(c + NBUF < NCHUNK)
            def _():
                pltpu.async_copy(
                    x_hbm.at[pl.ds(row0 + NBUF * CHUNK, CHUNK)],
                    xbuf.at[pl.ds(b * CHUNK, CHUNK)], isem[b])

    for b in range(NBUF):
        pltpu.make_async_copy(
            obuf.at[pl.ds(b * CHUNK, CHUNK)],
            out_hbm.at[pl.ds(base + b * CHUNK, CHUNK)], osem[b]).wait()


def kernel(x, params):
    scaler = jnp.array([1.0e12, 1.0e8, 1.0e4, 1.0, 10.0], dtype=jnp.float32)
    return _interp(x, (params / scaler).reshape(-1))
